# baseline (speedup 1.0000x reference)
import functools

import jax
import jax.numpy as jnp
import numpy as np
from jax import lax
from jax.experimental import pallas as pl
from jax.experimental.pallas import tpu as pltpu
from jax.experimental.pallas import tpu_sc as plsc

N = 10000
E = 320000
D = 128
NC = 2
NS = 16
NW = NC * NS
CH = 79
EPT = CH * 128
E_PAD = NW * EPT
CH0 = 107
CH1 = 51
ACC = 10240
STRIPE = ACC // NS
NB = ACC // 128
ROW_BLK = 2000
GRID = N // ROW_BLK

def _perm_mat():
    ri = lax.broadcasted_iota(jnp.int32, (128, 128), 0)
    jc = lax.broadcasted_iota(jnp.int32, (128, 128), 1)
    g = jc // 32
    r = jc % 32
    src = 32 * g + jnp.where(r % 2 == 0, r // 2, 16 + (r - 1) // 2)
    return (ri == src).astype(jnp.float32)


def _hist_body(dst_h, out_h, didx, hist):
    c = lax.axis_index("c")
    s = lax.axis_index("s")
    wid = c * NS + s
    pltpu.sync_copy(dst_h.at[wid], didx)

    def zero(i, carry):
        hist[pl.ds(i * 16, 16)] = jnp.zeros((16,), jnp.float32)
        return carry

    lax.fori_loop(0, ACC // 16, zero, 0)

    ones = jnp.ones((16,), jnp.float32)

    def chunk(j, carry):
        for k in range(8):
            idx = didx[j, pl.ds(k * 16, 16)]
            plsc.addupdate_scatter(hist, [idx], ones)
        return carry

    lax.fori_loop(0, CH, chunk, 0)
    pltpu.sync_copy(hist, out_h.at[wid])


@functools.cache
def _sc_kernels():
    mesh = plsc.VectorSubcoreMesh(
        core_axis_name="c", subcore_axis_name="s",
        num_cores=NC, num_subcores=NS)
    hist = pl.kernel(
        _hist_body,
        out_type=jax.ShapeDtypeStruct((NW, ACC), jnp.float32),
        mesh=mesh,
        compiler_params=pltpu.CompilerParams(needs_layout_passes=False),
        scratch_types=[
            pltpu.VMEM((CH, 128), jnp.int32),
            pltpu.VMEM((ACC,), jnp.float32),
        ],
    )
    agg = pl.kernel(
        _agg_body,
        out_type=jax.ShapeDtypeStruct((NC, ACC, D), jnp.float32),
        mesh=mesh,
        compiler_params=pltpu.CompilerParams(
            needs_layout_passes=False, use_tc_tiling_on_sc=False),
        scratch_types=[
            pltpu.VMEM((CH0, 128), jnp.int32),
            pltpu.VMEM((4, 128), jnp.int32),
            pltpu.VMEM((4, 128), jnp.int32),
            pltpu.VMEM((128, D // 2), jnp.int32),
            pltpu.VMEM((128, D // 2), jnp.int32),
            pltpu.VMEM((128, D), jnp.float32),
            pltpu.VMEM_SHARED((ACC, D), jnp.float32),
            pltpu.SemaphoreType.DMA,
            pltpu.SemaphoreType.DMA,
            pltpu.SemaphoreType.DMA,
        ],
    )
    return hist, agg


def _agg_body(y_h, pidxa_h, pidxb_h, z_h, out_h, pidx, sidxb, didxb,
              bb0, bb1, fbuf, acc, g0, g1, ssem):
    c = lax.axis_index("c")
    s = lax.axis_index("s")
    bbufs = (bb0, bb1)
    gs = (g0, g1)
    pltpu.sync_copy(z_h, acc.at[pl.ds(s * STRIPE, STRIPE)])

    def unpack(j, slot):
        for k in range(8):
            v = pidx[j, pl.ds(k * 16, 16)]
            sidxb[slot, pl.ds(k * 16, 16)] = lax.shift_right_logical(v, 14)
            didxb[slot, pl.ds(k * 16, 16)] = lax.bitwise_and(v, 16383)

    def convert(b):
        def row(r, carry):
            for k in range(4):
                v = bbufs[b][r, pl.ds(k * 16, 16)]
                lo = plsc.bitcast(v << 16, jnp.float32)
                hi = plsc.bitcast(v & jnp.int32(-65536), jnp.float32)
                fbuf[r, pl.ds(k * 32, 16)] = lo
                fbuf[r, pl.ds(k * 32 + 16, 16)] = hi
            return carry

        lax.fori_loop(0, 128, row, 0)

    def start_gather(jslot, b):
        pltpu.async_copy(y_h.at[sidxb.at[jslot]], bbufs[b], gs[b])

    def wait_gather(jslot, b):
        pltpu.make_async_copy(y_h.at[sidxb.at[jslot]], bbufs[b], gs[b]).wait()

    def start_scatter(jslot):
        pltpu.async_copy(fbuf, acc.at[didxb.at[jslot]], ssem, add=True)

    def wait_scatter(jslot):
        pltpu.make_async_copy(fbuf, acc.at[didxb.at[jslot]], ssem).wait()

    def pipeline(ch, slab_h):
        pltpu.sync_copy(slab_h.at[s], pidx.at[pl.ds(0, ch)])
        plsc.subcore_barrier()
        unpack(0, 0)
        unpack(1, 1)
        start_gather(0, 0)
        wait_gather(0, 0)
        unpack(2, 2)
        start_gather(1, 1)
        convert(0)
        start_scatter(0)

        def outer(g, carry):
            for bb in range(4):
                j = g * 4 + bb + 1
                js = (bb + 1) % 4
                b = (bb + 1) % 2
                wait_gather(js, b)
                unpack(j + 2, (js + 2) % 4)
                start_gather((js + 1) % 4, (b + 1) % 2)
                wait_scatter((js + 3) % 4)
                convert(b)
                start_scatter(js)
            return carry

        lax.fori_loop(0, (ch - 3) // 4, outer, 0)
        wait_gather(1, 1)
        start_gather(2, 0)
        wait_scatter(0)
        convert(1)
        start_scatter(1)
        wait_gather(2, 0)
        wait_scatter(1)
        convert(0)
        start_scatter(2)
        wait_scatter(2)

    @pl.when(c == 0)
    def _():
        pipeline(CH0, pidxa_h)

    @pl.when(c == 1)
    def _():
        pipeline(CH1, pidxb_h)

    plsc.subcore_barrier()
    pltpu.sync_copy(acc.at[pl.ds(s * STRIPE, STRIPE)],
                    out_h.at[c, pl.ds(s * STRIPE, STRIPE)])


def _dinv_body(h_ref, out_ref):
    deg = jnp.sum(h_ref[...], axis=0) + 1.0
    dinv = 1.0 / jnp.sqrt(deg)
    d2 = dinv.reshape(NB, 128)
    r_io = lax.broadcasted_iota(jnp.int32, (128, 128), 0)
    c_io = lax.broadcasted_iota(jnp.int32, (128, 128), 1)
    eye = (r_io == c_io).astype(jnp.float32)
    t = lax.dot_general(eye, d2, (((0,), (1,)), ((), ())),
                        preferred_element_type=jnp.float32)
    for i in range(NB):
        out_ref[i * 128:(i + 1) * 128, :] = jnp.broadcast_to(
            t[:, i:i + 1], (128, D))


_dinv = pl.pallas_call(
    _dinv_body,
    out_shape=jax.ShapeDtypeStruct((ACC, D), jnp.float32),
)


def _prep_body(x_ref, w_ref, dinv_ref, y_ref, ybf_ref):
    y = dinv_ref[...] * jnp.dot(
        x_ref[...], w_ref[...], preferred_element_type=jnp.float32)
    y_ref[...] = y
    ybf_ref[...] = jnp.dot(
        y, _perm_mat(), preferred_element_type=jnp.float32).astype(jnp.bfloat16)


_prep = pl.pallas_call(
    _prep_body,
    grid=(GRID,),
    in_specs=[
        pl.BlockSpec((ROW_BLK, D), lambda i: (i, 0)),
        pl.BlockSpec((D, D), lambda i: (0, 0)),
        pl.BlockSpec((ROW_BLK, D), lambda i: (i, 0)),
    ],
    out_specs=[pl.BlockSpec((ROW_BLK, D), lambda i: (i, 0)),
               pl.BlockSpec((ROW_BLK, D), lambda i: (i, 0))],
    out_shape=[jax.ShapeDtypeStruct((N, D), jnp.float32),
               jax.ShapeDtypeStruct((N, D), jnp.bfloat16)],
)

_BN_C = float(1.0 / np.sqrt(1.0 + 1e-5))


def _mid_body(a_ref, y_ref, dinv_ref, p_ref, w_ref, o_ref, obf_ref):
    dinv = dinv_ref[...]
    agg = a_ref[0] + a_ref[1] + y_ref[...]
    z = dinv * agg + p_ref[0:1, :]
    z = z * _BN_C * p_ref[1:2, :] + p_ref[2:3, :]
    h = jnp.maximum(z, 0.0)
    o = dinv * jnp.dot(h, w_ref[...], preferred_element_type=jnp.float32)
    o_ref[...] = o
    obf_ref[...] = jnp.dot(
        o, _perm_mat(), preferred_element_type=jnp.float32).astype(jnp.bfloat16)


_mid = pl.pallas_call(
    _mid_body,
    grid=(GRID,),
    in_specs=[
        pl.BlockSpec((NC, ROW_BLK, D), lambda i: (0, i, 0)),
        pl.BlockSpec((ROW_BLK, D), lambda i: (i, 0)),
        pl.BlockSpec((ROW_BLK, D), lambda i: (i, 0)),
        pl.BlockSpec((8, D), lambda i: (0, 0)),
        pl.BlockSpec((D, D), lambda i: (0, 0)),
    ],
    out_specs=[pl.BlockSpec((ROW_BLK, D), lambda i: (i, 0)),
               pl.BlockSpec((ROW_BLK, D), lambda i: (i, 0))],
    out_shape=[jax.ShapeDtypeStruct((N, D), jnp.float32),
               jax.ShapeDtypeStruct((N, D), jnp.bfloat16)],
)


def _final_body(a_ref, y_ref, dinv_ref, p_ref, o_ref):
    agg = a_ref[0] + a_ref[1] + y_ref[...]
    o_ref[...] = dinv_ref[...] * agg + p_ref[0:1, :]


_final = pl.pallas_call(
    _final_body,
    grid=(GRID,),
    in_specs=[
        pl.BlockSpec((NC, ROW_BLK, D), lambda i: (0, i, 0)),
        pl.BlockSpec((ROW_BLK, D), lambda i: (i, 0)),
        pl.BlockSpec((ROW_BLK, D), lambda i: (i, 0)),
        pl.BlockSpec((8, D), lambda i: (0, 0)),
    ],
    out_specs=pl.BlockSpec((ROW_BLK, D), lambda i: (i, 0)),
    out_shape=jax.ShapeDtypeStruct((N, D), jnp.float32),
)


def kernel(x, adj_t, W1, b1, g1, be1, W2, b2, g2, be2, W3, b3):
    src = adj_t[0].astype(jnp.int32)
    dst = adj_t[1].astype(jnp.int32)
    pad = E_PAD - E
    src_p = jnp.concatenate([src, jnp.zeros((pad,), jnp.int32)])
    dst_p = jnp.concatenate([dst, jnp.full((pad,), N, jnp.int32)])
    dst3 = dst_p.reshape(NW, CH, 128)
    packed = (src_p << 14) | dst_p
    cut = NS * CH0 * 128
    pidx_a = packed[:cut].reshape(NS, CH0, 128)
    pidx_b = packed[cut:].reshape(NS, CH1, 128)

    z128 = jnp.zeros((STRIPE, D), jnp.float32)

    _hist, _agg = _sc_kernels()
    hist = _hist(dst3)
    dinv_rep = _dinv(hist)

    p1 = jnp.zeros((8, D), jnp.float32).at[0].set(b1).at[1].set(g1).at[2].set(be1)
    p2 = jnp.zeros((8, D), jnp.float32).at[0].set(b2).at[1].set(g2).at[2].set(be2)
    p3 = jnp.zeros((8, D), jnp.float32).at[0].set(b3)

    def as_i32(ybf):
        return lax.bitcast_convert_type(ybf.reshape(N, D // 2, 2), jnp.int32)

    y1, y1b = _prep(x, W1, dinv_rep)
    a1 = _agg(as_i32(y1b), pidx_a, pidx_b, z128)
    y2, y2b = _mid(a1, y1, dinv_rep, p1, W2)
    a2 = _agg(as_i32(y2b), pidx_a, pidx_b, z128)
    y3, y3b = _mid(a2, y2, dinv_rep, p2, W3)
    a3 = _agg(as_i32(y3b), pidx_a, pidx_b, z128)
    out = _final(a3, y3, dinv_rep, p3)
    return out


# speedup vs baseline: 1.3515x; 1.3515x over previous
import functools

import jax
import jax.numpy as jnp
import numpy as np
from jax import lax
from jax.experimental import pallas as pl
from jax.experimental.pallas import tpu as pltpu
from jax.experimental.pallas import tpu_sc as plsc

N = 10000
E = 320000
D = 128
NC = 2
NS = 16
NW = NC * NS
CH = 79
EPT = CH * 128
E_PAD = NW * EPT
CH0 = 107
CH1 = 51
ACC = 10240
STRIPE = ACC // NS
NB = ACC // 128
ROW_BLK = 2000
GRID = N // ROW_BLK


def _hist_body(dst_h, out_h, didx, hist):
    c = lax.axis_index("c")
    s = lax.axis_index("s")
    wid = c * NS + s
    pltpu.sync_copy(dst_h.at[wid], didx)

    def zero(i, carry):
        hist[pl.ds(i * 16, 16)] = jnp.zeros((16,), jnp.float32)
        return carry

    lax.fori_loop(0, ACC // 16, zero, 0)

    ones = jnp.ones((16,), jnp.float32)

    def chunk(j, carry):
        for k in range(8):
            idx = didx[j, pl.ds(k * 16, 16)]
            plsc.addupdate_scatter(hist, [idx], ones)
        return carry

    lax.fori_loop(0, CH, chunk, 0)
    pltpu.sync_copy(hist, out_h.at[wid])


@functools.cache
def _sc_kernels():
    mesh = plsc.VectorSubcoreMesh(
        core_axis_name="c", subcore_axis_name="s",
        num_cores=NC, num_subcores=NS)
    hist = pl.kernel(
        _hist_body,
        out_type=jax.ShapeDtypeStruct((NW, ACC), jnp.float32),
        mesh=mesh,
        compiler_params=pltpu.CompilerParams(needs_layout_passes=False),
        scratch_types=[
            pltpu.VMEM((CH, 128), jnp.int32),
            pltpu.VMEM((ACC,), jnp.float32),
        ],
    )
    agg = pl.kernel(
        _agg_body,
        out_type=jax.ShapeDtypeStruct((NC, ACC, D), jnp.float32),
        mesh=mesh,
        compiler_params=pltpu.CompilerParams(needs_layout_passes=False),
        scratch_types=[
            pltpu.VMEM((CH0, 128), jnp.int32),
            pltpu.VMEM((4, 128), jnp.int32),
            pltpu.VMEM((4, 128), jnp.int32),
            pltpu.VMEM((128, D), jnp.float32),
            pltpu.VMEM((128, D), jnp.float32),
            pltpu.VMEM_SHARED((ACC, D), jnp.float32),
            pltpu.SemaphoreType.DMA,
            pltpu.SemaphoreType.DMA,
            pltpu.SemaphoreType.DMA,
            pltpu.SemaphoreType.DMA,
        ],
    )
    return hist, agg


def _agg_body(y_h, pidxa_h, pidxb_h, z_h, out_h, pidx, sidxb, didxb,
              b0, b1, acc, g0, g1, s0, s1):
    c = lax.axis_index("c")
    s = lax.axis_index("s")
    bufs = (b0, b1)
    gs = (g0, g1)
    ss = (s0, s1)
    pltpu.sync_copy(z_h, acc.at[pl.ds(s * STRIPE, STRIPE)])

    def unpack(j, slot):
        for k in range(8):
            v = pidx[j, pl.ds(k * 16, 16)]
            sidxb[slot, pl.ds(k * 16, 16)] = lax.shift_right_logical(v, 14)
            didxb[slot, pl.ds(k * 16, 16)] = lax.bitwise_and(v, 16383)

    def wait_gather(jslot, b):
        pltpu.make_async_copy(y_h.at[sidxb.at[jslot]], bufs[b], gs[b]).wait()

    def start_scatter(jslot, b):
        pltpu.async_copy(bufs[b], acc.at[didxb.at[jslot]], ss[b], add=True)

    def wait_scatter(jslot, b):
        pltpu.make_async_copy(bufs[b], acc.at[didxb.at[jslot]], ss[b]).wait()

    def pipeline(ch, slab_h):
        pltpu.sync_copy(slab_h.at[s], pidx.at[pl.ds(0, ch)])
        plsc.subcore_barrier()
        unpack(0, 0)
        unpack(1, 1)
        pltpu.async_copy(y_h.at[sidxb.at[0]], bufs[0], gs[0])
        wait_gather(0, 0)
        start_scatter(0, 0)
        unpack(2, 2)
        pltpu.async_copy(y_h.at[sidxb.at[1]], bufs[1], gs[1])

        def outer(g, carry):
            for b in range(4):
                j = g * 4 + b + 1
                js = (b + 1) % 4
                jb = (b + 1) % 2
                wait_scatter((js + 3) % 4, (jb + 1) % 2)
                wait_gather(js, jb)
                start_scatter(js, jb)
                unpack(j + 2, (js + 2) % 4)
                pltpu.async_copy(y_h.at[sidxb.at[(js + 1) % 4]],
                                 bufs[(jb + 1) % 2], gs[(jb + 1) % 2])
            return carry

        lax.fori_loop(0, (ch - 3) // 4, outer, 0)
        wait_scatter(0, 0)
        wait_gather(1, 1)
        start_scatter(1, 1)
        pltpu.async_copy(y_h.at[sidxb.at[2]], bufs[0], gs[0])
        wait_scatter(1, 1)
        wait_gather(2, 0)
        start_scatter(2, 0)
        wait_scatter(2, 0)

    @pl.when(c == 0)
    def _():
        pipeline(CH0, pidxa_h)

    @pl.when(c == 1)
    def _():
        pipeline(CH1, pidxb_h)

    plsc.subcore_barrier()
    pltpu.sync_copy(acc.at[pl.ds(s * STRIPE, STRIPE)],
                    out_h.at[c, pl.ds(s * STRIPE, STRIPE)])


def _dinv_body(h_ref, out_ref):
    deg = jnp.sum(h_ref[...], axis=0) + 1.0
    dinv = 1.0 / jnp.sqrt(deg)
    d2 = dinv.reshape(NB, 128)
    r_io = lax.broadcasted_iota(jnp.int32, (128, 128), 0)
    c_io = lax.broadcasted_iota(jnp.int32, (128, 128), 1)
    eye = (r_io == c_io).astype(jnp.float32)
    t = lax.dot_general(eye, d2, (((0,), (1,)), ((), ())),
                        preferred_element_type=jnp.float32)
    for i in range(NB):
        out_ref[i * 128:(i + 1) * 128, :] = jnp.broadcast_to(
            t[:, i:i + 1], (128, D))


_dinv = pl.pallas_call(
    _dinv_body,
    out_shape=jax.ShapeDtypeStruct((ACC, D), jnp.float32),
)


def _prep_body(x_ref, w_ref, dinv_ref, y_ref):
    y_ref[...] = dinv_ref[...] * jnp.dot(
        x_ref[...], w_ref[...], preferred_element_type=jnp.float32)


_prep = pl.pallas_call(
    _prep_body,
    grid=(GRID,),
    in_specs=[
        pl.BlockSpec((ROW_BLK, D), lambda i: (i, 0)),
        pl.BlockSpec((D, D), lambda i: (0, 0)),
        pl.BlockSpec((ROW_BLK, D), lambda i: (i, 0)),
    ],
    out_specs=pl.BlockSpec((ROW_BLK, D), lambda i: (i, 0)),
    out_shape=jax.ShapeDtypeStruct((N, D), jnp.float32),
)

_BN_C = float(1.0 / np.sqrt(1.0 + 1e-5))


def _mid_body(a_ref, y_ref, dinv_ref, p_ref, w_ref, o_ref):
    dinv = dinv_ref[...]
    agg = a_ref[0] + a_ref[1] + y_ref[...]
    z = dinv * agg + p_ref[0:1, :]
    z = z * _BN_C * p_ref[1:2, :] + p_ref[2:3, :]
    h = jnp.maximum(z, 0.0)
    o_ref[...] = dinv * jnp.dot(
        h, w_ref[...], preferred_element_type=jnp.float32)


_mid = pl.pallas_call(
    _mid_body,
    grid=(GRID,),
    in_specs=[
        pl.BlockSpec((NC, ROW_BLK, D), lambda i: (0, i, 0)),
        pl.BlockSpec((ROW_BLK, D), lambda i: (i, 0)),
        pl.BlockSpec((ROW_BLK, D), lambda i: (i, 0)),
        pl.BlockSpec((8, D), lambda i: (0, 0)),
        pl.BlockSpec((D, D), lambda i: (0, 0)),
    ],
    out_specs=pl.BlockSpec((ROW_BLK, D), lambda i: (i, 0)),
    out_shape=jax.ShapeDtypeStruct((N, D), jnp.float32),
)


def _final_body(a_ref, y_ref, dinv_ref, p_ref, o_ref):
    agg = a_ref[0] + a_ref[1] + y_ref[...]
    o_ref[...] = dinv_ref[...] * agg + p_ref[0:1, :]


_final = pl.pallas_call(
    _final_body,
    grid=(GRID,),
    in_specs=[
        pl.BlockSpec((NC, ROW_BLK, D), lambda i: (0, i, 0)),
        pl.BlockSpec((ROW_BLK, D), lambda i: (i, 0)),
        pl.BlockSpec((ROW_BLK, D), lambda i: (i, 0)),
        pl.BlockSpec((8, D), lambda i: (0, 0)),
    ],
    out_specs=pl.BlockSpec((ROW_BLK, D), lambda i: (i, 0)),
    out_shape=jax.ShapeDtypeStruct((N, D), jnp.float32),
)


def kernel(x, adj_t, W1, b1, g1, be1, W2, b2, g2, be2, W3, b3):
    src = adj_t[0].astype(jnp.int32)
    dst = adj_t[1].astype(jnp.int32)
    pad = E_PAD - E
    src_p = jnp.concatenate([src, jnp.zeros((pad,), jnp.int32)])
    dst_p = jnp.concatenate([dst, jnp.full((pad,), N, jnp.int32)])
    dst3 = dst_p.reshape(NW, CH, 128)
    packed = (src_p << 14) | dst_p
    cut = NS * CH0 * 128
    pidx_a = packed[:cut].reshape(NS, CH0, 128)
    pidx_b = packed[cut:].reshape(NS, CH1, 128)

    z128 = jnp.zeros((STRIPE, D), jnp.float32)

    _hist, _agg = _sc_kernels()
    hist = _hist(dst3)
    dinv_rep = _dinv(hist)

    p1 = jnp.zeros((8, D), jnp.float32).at[0].set(b1).at[1].set(g1).at[2].set(be1)
    p2 = jnp.zeros((8, D), jnp.float32).at[0].set(b2).at[1].set(g2).at[2].set(be2)
    p3 = jnp.zeros((8, D), jnp.float32).at[0].set(b3)

    y1 = _prep(x, W1, dinv_rep)
    a1 = _agg(y1, pidx_a, pidx_b, z128)
    y2 = _mid(a1, y1, dinv_rep, p1, W2)
    a2 = _agg(y2, pidx_a, pidx_b, z128)
    y3 = _mid(a2, y2, dinv_rep, p2, W3)
    a3 = _agg(y3, pidx_a, pidx_b, z128)
    out = _final(a3, y3, dinv_rep, p3)
    return out
